# baseline (speedup 1.0000x reference)
import functools

import jax
import jax.numpy as jnp
from jax.experimental import pallas as pl
from jax.experimental.pallas import tpu as pltpu

_NCHUNK = 8


def _copy_body(emb_ref, in_hbm, out_hbm, bcast_vmem, in_sems, emb_sem):
    batch = in_hbm.shape[0]
    chunk = batch // _NCHUNK
    copies = []
    for i in range(_NCHUNK):
        cp = pltpu.make_async_copy(
            in_hbm.at[pl.ds(i * chunk, chunk)],
            out_hbm.at[pl.ds(i * chunk, chunk), pl.ds(5, 200)],
            in_sems.at[i],
        )
        cp.start()
        copies.append(cp)

    bcast_vmem[...] = jnp.broadcast_to(
        emb_ref[...][None, :, :], (batch, 5, emb_ref.shape[1])
    )
    emb_cp = pltpu.make_async_copy(
        bcast_vmem, out_hbm.at[:, pl.ds(0, 5)], emb_sem
    )
    emb_cp.start()

    for cp in copies:
        cp.wait()
    emb_cp.wait()


@jax.jit
def kernel(inputs, emb_table):
    batch, rows, dim = inputs.shape
    n_emb = emb_table.shape[0]
    out_shape = jax.ShapeDtypeStruct((batch, rows + n_emb, dim), inputs.dtype)
    return pl.pallas_call(
        _copy_body,
        out_shape=out_shape,
        in_specs=[
            pl.BlockSpec(memory_space=pltpu.VMEM),
            pl.BlockSpec(memory_space=pltpu.MemorySpace.HBM),
        ],
        out_specs=pl.BlockSpec(memory_space=pltpu.MemorySpace.HBM),
        scratch_shapes=[
            pltpu.VMEM((batch, n_emb, dim), inputs.dtype),
            pltpu.SemaphoreType.DMA((_NCHUNK,)),
            pltpu.SemaphoreType.DMA,
        ],
    )(emb_table, inputs)


# speedup vs baseline: 21.9356x; 21.9356x over previous
import jax
import jax.numpy as jnp
from jax.experimental import pallas as pl
from jax.experimental.pallas import tpu as pltpu

_BBLK = 16


def _body(emb_ref, in_ref, out_ref):
    nb, ne, dim = out_ref.shape[0], emb_ref.shape[0], emb_ref.shape[1]
    out_ref[:, ne:, :] = in_ref[...]
    out_ref[:, :ne, :] = jnp.broadcast_to(emb_ref[...][None, :, :], (nb, ne, dim))


@jax.jit
def kernel(inputs, emb_table):
    batch, rows, dim = inputs.shape
    n_emb = emb_table.shape[0]
    out_shape = jax.ShapeDtypeStruct((batch, rows + n_emb, dim), inputs.dtype)
    grid = (batch // _BBLK,)
    return pl.pallas_call(
        _body,
        out_shape=out_shape,
        grid=grid,
        in_specs=[
            pl.BlockSpec((n_emb, dim), lambda b: (0, 0)),
            pl.BlockSpec((_BBLK, rows, dim), lambda b: (b, 0, 0)),
        ],
        out_specs=pl.BlockSpec((_BBLK, rows + n_emb, dim), lambda b: (b, 0, 0)),
    )(emb_table, inputs)


# speedup vs baseline: 23.6848x; 1.0797x over previous
import jax
import jax.numpy as jnp
from jax.experimental import pallas as pl
from jax.experimental.pallas import tpu as pltpu

_BBLK = 32


def _body(emb_ref, in_ref, out_ref):
    nb, ne, dim = out_ref.shape[0], emb_ref.shape[0], emb_ref.shape[1]
    out_ref[:, ne:, :] = in_ref[...]
    out_ref[:, :ne, :] = jnp.broadcast_to(emb_ref[...][None, :, :], (nb, ne, dim))


@jax.jit
def kernel(inputs, emb_table):
    batch, rows, dim = inputs.shape
    n_emb = emb_table.shape[0]
    out_shape = jax.ShapeDtypeStruct((batch, rows + n_emb, dim), inputs.dtype)
    grid = (batch // _BBLK,)
    return pl.pallas_call(
        _body,
        out_shape=out_shape,
        grid=grid,
        in_specs=[
            pl.BlockSpec((n_emb, dim), lambda b: (0, 0)),
            pl.BlockSpec((_BBLK, rows, dim), lambda b: (b, 0, 0)),
        ],
        out_specs=pl.BlockSpec((_BBLK, rows + n_emb, dim), lambda b: (b, 0, 0)),
    )(emb_table, inputs)


# speedup vs baseline: 24.0713x; 1.0163x over previous
import jax
import jax.numpy as jnp
from jax.experimental import pallas as pl
from jax.experimental.pallas import tpu as pltpu

_BBLK = 64


def _body(emb_ref, in_ref, out_ref):
    nb, ne, dim = out_ref.shape[0], emb_ref.shape[0], emb_ref.shape[1]
    out_ref[:, ne:, :] = in_ref[...]
    out_ref[:, :ne, :] = jnp.broadcast_to(emb_ref[...][None, :, :], (nb, ne, dim))


@jax.jit
def kernel(inputs, emb_table):
    batch, rows, dim = inputs.shape
    n_emb = emb_table.shape[0]
    out_shape = jax.ShapeDtypeStruct((batch, rows + n_emb, dim), inputs.dtype)
    grid = (batch // _BBLK,)
    return pl.pallas_call(
        _body,
        out_shape=out_shape,
        grid=grid,
        in_specs=[
            pl.BlockSpec((n_emb, dim), lambda b: (0, 0)),
            pl.BlockSpec((_BBLK, rows, dim), lambda b: (b, 0, 0)),
        ],
        out_specs=pl.BlockSpec((_BBLK, rows + n_emb, dim), lambda b: (b, 0, 0)),
    )(emb_table, inputs)
